# Initial kernel scaffold; baseline (speedup 1.0000x reference)
import jax
import jax.numpy as jnp
from jax.experimental import pallas as pl


def kernel(x, edge_index, offsets, W, b):
    raise NotImplementedError("write your pallas kernel here")


# speedup vs baseline: 3.7070x; 3.7070x over previous
import functools

import jax
import jax.numpy as jnp
from jax import lax
from jax.experimental import pallas as pl
from jax.experimental.pallas import tpu as pltpu
from jax.experimental.pallas import tpu_sc as plsc

NC = 2
NS = 16
CHUNK = 128


def _round_up(a, m):
    return (a + m - 1) // m * m


@functools.partial(jax.jit, static_argnames=("bm",))
def _tc_matmul(x, w_flat, bm=400):
    n, in_ch = x.shape
    _, cols = w_flat.shape

    def body(x_ref, w_ref, o_ref):
        o_ref[...] = jnp.dot(x_ref[...], w_ref[...],
                             preferred_element_type=jnp.float32)

    return pl.pallas_call(
        body,
        grid=(pl.cdiv(n, bm),),
        in_specs=[
            pl.BlockSpec((bm, in_ch), lambda i: (i, 0)),
            pl.BlockSpec((in_ch, cols), lambda i: (0, 0)),
        ],
        out_specs=pl.BlockSpec((bm, cols), lambda i: (i, 0)),
        out_shape=jax.ShapeDtypeStruct((n, cols), jnp.float32),
    )(x, w_flat)


@functools.partial(jax.jit, static_argnames=("npad", "ew", "nchunks", "out_ch"))
def _sc_scatter(z_rows, row_idx, dst_idx, zeros_init, *, npad, ew,
                nchunks, out_ch):
    mesh = plsc.VectorSubcoreMesh(core_axis_name="c", subcore_axis_name="s",
                                  num_cores=NC, num_subcores=NS)
    rpt = npad // NS

    @functools.partial(
        pl.kernel,
        out_type=jax.ShapeDtypeStruct((NC * npad, out_ch), jnp.float32),
        mesh=mesh,
        scratch_types=[
            pltpu.VMEM((CHUNK,), jnp.int32),
            pltpu.VMEM((CHUNK,), jnp.int32),
            pltpu.VMEM((CHUNK, out_ch), jnp.float32),
            pltpu.VMEM_SHARED((npad, out_ch), jnp.float32),
        ],
    )
    def sc_fn(z_hbm, ridx_hbm, didx_hbm, zeros_hbm, out_hbm,
              ridx_v, didx_v, rows_v, acc):
        cid = lax.axis_index("c")
        sid = lax.axis_index("s")
        wid = sid * NC + cid
        pltpu.sync_copy(zeros_hbm.at[pl.ds(sid * rpt, rpt)],
                        acc.at[pl.ds(sid * rpt, rpt)])
        plsc.subcore_barrier()

        base = wid * ew

        def body(g, carry):
            off = base + g * CHUNK
            pltpu.sync_copy(ridx_hbm.at[pl.ds(off, CHUNK)], ridx_v)
            pltpu.sync_copy(didx_hbm.at[pl.ds(off, CHUNK)], didx_v)
            pltpu.sync_copy(z_hbm.at[ridx_v], rows_v)
            pltpu.sync_copy(rows_v, acc.at[didx_v], add=True)
            return carry

        lax.fori_loop(0, nchunks, body, 0)
        plsc.subcore_barrier()
        pltpu.sync_copy(acc.at[pl.ds(sid * rpt, rpt)],
                        out_hbm.at[pl.ds(cid * npad + sid * rpt, rpt)])

    return sc_fn(z_rows, row_idx, dst_idx, zeros_init)


def kernel(x, edge_index, offsets, W, b):
    n, in_ch = x.shape
    kvol, _, out_ch = W.shape
    e = edge_index.shape[1]

    w_flat = jnp.transpose(W, (1, 0, 2)).reshape(in_ch, kvol * out_ch)
    z = _tc_matmul(x, w_flat)
    z_rows = z.reshape(n * kvol, out_ch)

    src = edge_index[0].astype(jnp.int32)
    dst = edge_index[1].astype(jnp.int32)
    row_idx = src * kvol + offsets.astype(jnp.int32)

    nw = NC * NS
    ew = _round_up(pl.cdiv(e, nw), CHUNK)
    ep = ew * nw
    nchunks = ew // CHUNK
    npad = _round_up(n + 1, NS * 8)

    pad = ep - e
    row_idx = jnp.concatenate([row_idx, jnp.zeros((pad,), jnp.int32)])
    dst_pad = jnp.concatenate([dst, jnp.full((pad,), n, jnp.int32)])
    zeros_init = jnp.zeros((npad, out_ch), jnp.float32)

    partials = _sc_scatter(z_rows, row_idx, dst_pad, zeros_init,
                           npad=npad, ew=ew, nchunks=nchunks, out_ch=out_ch)
    return partials[:n] + partials[npad:npad + n] + b
